# baseline (speedup 1.0000x reference)
import functools

import jax
import jax.numpy as jnp
from jax import lax
from jax.experimental import pallas as pl
from jax.experimental.pallas import tpu as pltpu
from jax.experimental.pallas import tpu_sc as plsc

H = 2048
E = 64
EPS = 1e-06
TILE = 512
NW = 32
LANES = 16


def _logits_body(x_ref, w_ref, scale_ref, lt_ref):
    x = x_ref[...]
    h = x * lax.rsqrt(jnp.mean(x * x, axis=1, keepdims=True) + EPS)
    h = h * (scale_ref[...] * (float(H) ** -0.5))
    logits = lax.dot_general(
        h, w_ref[...], (((1,), (1,)), ((), ())),
        preferred_element_type=jnp.float32)
    lt_ref[...] = logits.T


def _tc_logits(hidden_states, W, scale):
    T = hidden_states.shape[0]
    return pl.pallas_call(
        _logits_body,
        grid=(T // TILE,),
        in_specs=[
            pl.BlockSpec((TILE, H), lambda i: (i, 0)),
            pl.BlockSpec((E, H), lambda i: (0, 0)),
            pl.BlockSpec((1, H), lambda i: (0, 0)),
        ],
        out_specs=pl.BlockSpec((E, TILE), lambda i: (0, i)),
        out_shape=jax.ShapeDtypeStruct((E, T), jnp.float32),
        compiler_params=pltpu.CompilerParams(
            dimension_semantics=("arbitrary",),
        ),
    )(hidden_states, W, scale.reshape(1, H))


def _make_sc_router(T):
    C = T // NW
    G = C // LANES
    mesh = plsc.VectorSubcoreMesh(core_axis_name="c", subcore_axis_name="s")

    @functools.partial(
        pl.kernel, mesh=mesh,
        out_type=[
            jax.ShapeDtypeStruct((T,), jnp.float32),
            jax.ShapeDtypeStruct((T,), jnp.float32),
            jax.ShapeDtypeStruct((T,), jnp.int32),
            jax.ShapeDtypeStruct((T,), jnp.int32),
        ],
        scratch_types=[
            pltpu.VMEM((E, C), jnp.float32),
            pltpu.VMEM((E, LANES), jnp.float32),
            pltpu.VMEM((C,), jnp.float32),
            pltpu.VMEM((C,), jnp.float32),
            pltpu.VMEM((C,), jnp.int32),
            pltpu.VMEM((C,), jnp.int32),
        ],
    )
    def sc_router(lt_hbm, pes_hbm, w1_hbm, w2_hbm, i1_hbm, i2_hbm,
                  loc, pes_v, w1_v, w2_v, i1_v, i2_v):
        wid = lax.axis_index("s") * 2 + lax.axis_index("c")
        base = wid * C
        pltpu.sync_copy(pes_hbm, pes_v)
        pltpu.sync_copy(lt_hbm.at[:, pl.ds(base, C)], loc)

        def group(g, carry):
            off = g * LANES
            m1 = jnp.full((LANES,), -jnp.inf, jnp.float32)
            m2 = jnp.full((LANES,), -jnp.inf, jnp.float32)
            i1 = jnp.zeros((LANES,), jnp.int32)
            i2 = jnp.zeros((LANES,), jnp.int32)
            s1 = jnp.zeros((LANES,), jnp.float32)
            s2 = jnp.zeros((LANES,), jnp.float32)
            for e in range(E):
                v = loc[e, pl.ds(off, LANES)]
                pv = pes_v[e, :]
                ev = jnp.full((LANES,), e, jnp.int32)
                gt1 = v > m1
                gt2 = v > m2
                i2 = jnp.where(gt1, i1, jnp.where(gt2, ev, i2))
                m2 = jnp.where(gt1, m1, jnp.where(gt2, v, m2))
                s2 = jnp.where(gt1, s1, jnp.where(gt2, pv, s2))
                i1 = jnp.where(gt1, ev, i1)
                m1 = jnp.where(gt1, v, m1)
                s1 = jnp.where(gt1, pv, s1)
            t = jnp.exp(m2 - m1)
            denom = 1.0 + t
            w1_v[pl.ds(off, LANES)] = s1 / denom
            w2_v[pl.ds(off, LANES)] = (t / denom) * s2
            i1_v[pl.ds(off, LANES)] = i1
            i2_v[pl.ds(off, LANES)] = i2
            return carry

        lax.fori_loop(0, G, group, 0)
        pltpu.sync_copy(w1_v, w1_hbm.at[pl.ds(base, C)])
        pltpu.sync_copy(w2_v, w2_hbm.at[pl.ds(base, C)])
        pltpu.sync_copy(i1_v, i1_hbm.at[pl.ds(base, C)])
        pltpu.sync_copy(i2_v, i2_hbm.at[pl.ds(base, C)])

    return sc_router


def kernel(hidden_states, W, scale, per_expert_scale):
    T = hidden_states.shape[0]
    lt = _tc_logits(hidden_states, W, scale)
    pes_b = jnp.broadcast_to(per_expert_scale[:, None], (E, LANES))
    w1, w2, i1, i2 = _make_sc_router(T)(lt, pes_b)
    top_w = jnp.stack([w1, w2], axis=1)
    top_i = jnp.stack([i1, i2], axis=1)
    return (top_w, top_i)


# speedup vs baseline: 1.3085x; 1.3085x over previous
import jax
import jax.numpy as jnp
from jax.experimental import pallas as pl
from jax.experimental.pallas import tpu as pltpu

H = 2048
E = 64
EPS = 1e-06
TILE = 2048


def _router_body(x_ref, w_ref, scale_ref, pes_ref, tw_ref, ti_ref):
    x = x_ref[...]
    h = x * jax.lax.rsqrt(jnp.mean(x * x, axis=1, keepdims=True) + EPS)
    h = h * (scale_ref[...] * (float(H) ** -0.5))
    logits = jax.lax.dot_general(
        h, w_ref[...], (((1,), (1,)), ((), ())),
        preferred_element_type=jnp.float32)

    idx = jax.lax.broadcasted_iota(jnp.int32, logits.shape, 1)
    m1 = jnp.max(logits, axis=1, keepdims=True)
    i1 = jnp.min(jnp.where(logits == m1, idx, E), axis=1, keepdims=True)
    masked = jnp.where(idx == i1, jnp.finfo(jnp.float32).min, logits)
    m2 = jnp.max(masked, axis=1, keepdims=True)
    i2 = jnp.min(jnp.where(masked == m2, idx, E), axis=1, keepdims=True)

    e = jnp.exp(m2 - m1)
    denom = 1.0 + e
    pes = pes_ref[...]
    s1 = jnp.sum(jnp.where(idx == i1, pes, 0.0), axis=1, keepdims=True)
    s2 = jnp.sum(jnp.where(idx == i2, pes, 0.0), axis=1, keepdims=True)
    tw_ref[:, 0:1] = s1 / denom
    tw_ref[:, 1:2] = (e / denom) * s2
    ti_ref[:, 0:1] = i1
    ti_ref[:, 1:2] = i2


def kernel(hidden_states, W, scale, per_expert_scale):
    T = hidden_states.shape[0]
    grid = (T // TILE,)
    scale2d = scale.reshape(1, H)
    pes2d = per_expert_scale.reshape(1, E)
    top_w, top_i = pl.pallas_call(
        _router_body,
        grid=grid,
        in_specs=[
            pl.BlockSpec((TILE, H), lambda i: (i, 0)),
            pl.BlockSpec((E, H), lambda i: (0, 0)),
            pl.BlockSpec((1, H), lambda i: (0, 0)),
            pl.BlockSpec((1, E), lambda i: (0, 0)),
        ],
        out_specs=[
            pl.BlockSpec((TILE, 2), lambda i: (i, 0)),
            pl.BlockSpec((TILE, 2), lambda i: (i, 0)),
        ],
        out_shape=[
            jax.ShapeDtypeStruct((T, 2), jnp.float32),
            jax.ShapeDtypeStruct((T, 2), jnp.int32),
        ],
        compiler_params=pltpu.CompilerParams(
            dimension_semantics=("arbitrary",),
        ),
    )(hidden_states, W, scale2d, pes2d)
    return (top_w, top_i)
